# baseline (speedup 1.0000x reference)
import functools

import jax
import jax.numpy as jnp
from jax import lax
from jax.experimental import pallas as pl
from jax.experimental.pallas import tpu as pltpu
from jax.experimental.pallas import tpu_sc as plsc

N = 10000
D = 128
NE = 4
NG = 16

NC = 2
NS = 16
TILES = NC * NS
CH = 80
ROWS_PER_TILE = 640
RCH = ROWS_PER_TILE // CH
N_PAD = ROWS_PER_TILE * NS

RB = 1000
GRID = N // RB
NB_PAD = 10240


def _sc_mesh():
    return plsc.VectorSubcoreMesh(core_axis_name="c", subcore_axis_name="s",
                                  num_cores=NC, num_subcores=NS)


def _sc_pipeline(n, T, s_wait, id_issue, id_wait, s_issue,
                 g_issue=None, g_wait=None, is_issue=None, is_wait=None):
    gather = g_issue is not None

    def head_step(k):
        j = k % 4
        id_issue(k, j)
        if gather:
            is_wait(k, j)
            g_issue(k, j)
        if k >= 2:
            jd = (k + 2) % 4
            if gather:
                g_wait(k - 2, jd)
                is_issue(k + 2, jd)
            id_wait(k - 2, jd)
            s_issue(k - 2, jd)
        elif gather:
            is_issue(k + 2, (k + 2) % 4)

    if gather:
        is_issue(0, 0)
        is_issue(1, 1)
    for k in range(4):
        head_step(k)

    def body(t, carry):
        for dlt in range(4):
            k = 4 * t + dlt
            j = dlt
            jd = (dlt + 2) % 4
            s_wait(k - 4, j)
            id_issue(k, j)
            if gather:
                is_wait(k, j)
                g_issue(k, j)
                g_wait(k - 2, jd)
                is_issue(k + 2, jd)
            id_wait(k - 2, jd)
            s_issue(k - 2, jd)
        return carry

    lax.fori_loop(1, T, body, 0)
    s_wait(n - 4, 0)
    if gather:
        g_wait(n - 2, 2)
    id_wait(n - 2, 2)
    s_issue(n - 2, 2)
    s_wait(n - 3, 1)
    if gather:
        g_wait(n - 1, 3)
    id_wait(n - 1, 3)
    s_issue(n - 1, 3)
    if gather:
        is_wait(n, 0)
        is_wait(n + 1, 1)
    s_wait(n - 2, 2)
    s_wait(n - 1, 3)


def _make_ring_ops(cpt, w, v_hbm, src_hbm, dst_hbm, rbs, sbu, dbu,
                   sgs, sis, sds, sss, acc_sh):
    n = cpt

    def is_issue(k, j):
        kk = k if isinstance(k, int) and k < n else lax.min(k, n - 1) if not isinstance(k, int) else min(k, n - 1)
        pltpu.async_copy(src_hbm.at[pl.ds((w * cpt + kk) * CH, CH)],
                         sbu[j], sis[j])

    def is_wait(k, j):
        kk = min(k, n - 1) if isinstance(k, int) else lax.min(k, n - 1)
        pltpu.make_async_copy(src_hbm.at[pl.ds((w * cpt + kk) * CH, CH)],
                              sbu[j], sis[j]).wait()

    def id_issue(k, j):
        pltpu.async_copy(dst_hbm.at[pl.ds((w * cpt + k) * CH, CH)],
                         dbu[j], sds[j])

    def id_wait(k, j):
        pltpu.make_async_copy(dst_hbm.at[pl.ds((w * cpt + k) * CH, CH)],
                              dbu[j], sds[j]).wait()

    def g_issue(k, j):
        pltpu.async_copy(v_hbm.at[sbu[j]], rbs[j], sgs[j])

    def g_wait(k, j):
        pltpu.make_async_copy(v_hbm.at[sbu[j]], rbs[j], sgs[j]).wait()

    def s_issue(k, j, src_buf=None):
        pltpu.async_copy(rbs[j] if src_buf is None else src_buf,
                         acc_sh.at[dbu[j]], sss[j], add=True)

    def s_wait(k, j, src_buf=None):
        pltpu.make_async_copy(rbs[j] if src_buf is None else src_buf,
                              acc_sh.at[dbu[j]], sss[j]).wait()

    return dict(is_issue=is_issue, is_wait=is_wait, id_issue=id_issue,
                id_wait=id_wait, g_issue=g_issue, g_wait=g_wait,
                s_issue=s_issue, s_wait=s_wait)


_SC_SCRATCH = (
    [pltpu.VMEM((CH, D), jnp.float32) for _ in range(4)]
    + [pltpu.VMEM((CH,), jnp.int32) for _ in range(8)]
    + [pltpu.SemaphoreType.DMA for _ in range(16)]
    + [pltpu.VMEM_SHARED((N_PAD, D), jnp.float32)]
)


def _make_agg_h(cpt):
    T = cpt // 4
    assert cpt % 4 == 0 and cpt >= 8

    @functools.partial(
        pl.kernel,
        out_type=[
            jax.ShapeDtypeStruct((NC * N_PAD, D), jnp.float32),
            jax.ShapeDtypeStruct((NC * N_PAD, D), jnp.float32),
        ],
        mesh=_sc_mesh(),
        scratch_types=list(_SC_SCRATCH),
    )
    def agg(v_hbm, src_hbm, dst_hbm, zrow_hbm, ones_hbm,
            out_hbm, deg_hbm,
            rb0, rb1, rb2, rb3, sb0, sb1, sb2, sb3, db0, db1, db2, db3,
            sg0, sg1, sg2, sg3, si0, si1, si2, si3,
            sd0, sd1, sd2, sd3, ss0, ss1, ss2, ss3, acc_sh):
        c = lax.axis_index("c")
        s = lax.axis_index("s")
        w = s * NC + (1 - c)
        r0 = s * ROWS_PER_TILE
        o0 = c * N_PAD + r0
        rbs = [rb0, rb1, rb2, rb3]
        ops = _make_ring_ops(cpt, w, v_hbm, src_hbm, dst_hbm, rbs,
                             [sb0, sb1, sb2, sb3], [db0, db1, db2, db3],
                             [sg0, sg1, sg2, sg3], [si0, si1, si2, si3],
                             [sd0, sd1, sd2, sd3], [ss0, ss1, ss2, ss3],
                             acc_sh)

        def zero_acc():
            pltpu.sync_copy(zrow_hbm, rbs[0])
            for j in range(RCH):
                pltpu.sync_copy(rbs[0], acc_sh.at[pl.ds(r0 + j * CH, CH)])

        def writeout(dst_flat):
            for j in range(RCH):
                pltpu.sync_copy(acc_sh.at[pl.ds(r0 + j * CH, CH)], rbs[0])
                pltpu.sync_copy(rbs[0], dst_flat.at[pl.ds(o0 + j * CH, CH)])

        zero_acc()
        plsc.subcore_barrier()
        _sc_pipeline(cpt, T, ops["s_wait"], ops["id_issue"], ops["id_wait"],
                     ops["s_issue"], ops["g_issue"], ops["g_wait"],
                     ops["is_issue"], ops["is_wait"])
        plsc.subcore_barrier()
        writeout(out_hbm)

        zero_acc()
        pltpu.sync_copy(ones_hbm, rb1)
        plsc.subcore_barrier()

        def s_issue1(k, j):
            ops["s_issue"](k, j, src_buf=rb1)

        def s_wait1(k, j):
            ops["s_wait"](k, j, src_buf=rb1)

        _sc_pipeline(cpt, T, s_wait1, ops["id_issue"], ops["id_wait"],
                     s_issue1)
        plsc.subcore_barrier()
        writeout(deg_hbm)

    return agg


def _make_agg_experts(cpt):
    T = cpt // 4
    assert cpt % 4 == 0 and cpt >= 8

    @functools.partial(
        pl.kernel,
        out_type=[jax.ShapeDtypeStruct((NC * N_PAD, D), jnp.float32)
                  for _ in range(NE)],
        mesh=_sc_mesh(),
        scratch_types=list(_SC_SCRATCH),
    )
    def agg(v0_hbm, v1_hbm, v2_hbm, v3_hbm, src_hbm, dst_hbm, zrow_hbm,
            o0_hbm, o1_hbm, o2_hbm, o3_hbm,
            rb0, rb1, rb2, rb3, sb0, sb1, sb2, sb3, db0, db1, db2, db3,
            sg0, sg1, sg2, sg3, si0, si1, si2, si3,
            sd0, sd1, sd2, sd3, ss0, ss1, ss2, ss3, acc_sh):
        c = lax.axis_index("c")
        s = lax.axis_index("s")
        w = s * NC + (1 - c)
        r0 = s * ROWS_PER_TILE
        o0 = c * N_PAD + r0
        rbs = [rb0, rb1, rb2, rb3]
        vs = [v0_hbm, v1_hbm, v2_hbm, v3_hbm]
        os_ = [o0_hbm, o1_hbm, o2_hbm, o3_hbm]
        for e in range(NE):
            ops = _make_ring_ops(cpt, w, vs[e], src_hbm, dst_hbm, rbs,
                                 [sb0, sb1, sb2, sb3], [db0, db1, db2, db3],
                                 [sg0, sg1, sg2, sg3], [si0, si1, si2, si3],
                                 [sd0, sd1, sd2, sd3], [ss0, ss1, ss2, ss3],
                                 acc_sh)
            pltpu.sync_copy(zrow_hbm, rbs[0])
            for j in range(RCH):
                pltpu.sync_copy(rbs[0], acc_sh.at[pl.ds(r0 + j * CH, CH)])
            plsc.subcore_barrier()
            _sc_pipeline(cpt, T, ops["s_wait"], ops["id_issue"],
                         ops["id_wait"], ops["s_issue"], ops["g_issue"],
                         ops["g_wait"], ops["is_issue"], ops["is_wait"])
            plsc.subcore_barrier()
            for j in range(RCH):
                pltpu.sync_copy(acc_sh.at[pl.ds(r0 + j * CH, CH)], rbs[0])
                pltpu.sync_copy(rbs[0], os_[e].at[pl.ds(o0 + j * CH, CH)])

    return agg


def _encoder_body(x_ref, w_ref, b_ref, bfull_ref, bblk_ref, cent_ref,
                  h_ref, p_ref):
    h = jnp.dot(x_ref[...], w_ref[...], preferred_element_type=jnp.float32)
    h_ref[...] = jnp.maximum(h + b_ref[...], 0.0)
    bf = bfull_ref[...]
    bb = bblk_ref[...]
    inv_logn = 1.0 / jnp.log(jnp.float32(N))
    logn = jnp.zeros((RB, 1), jnp.float32)
    for g in range(NG):
        cnt = jnp.sum(jnp.where(bf == g, 1.0, 0.0))
        lg = jnp.log(jnp.maximum(cnt, 1.0)) * inv_logn
        logn = logn + jnp.where(bb == g, lg, 0.0)
    dlt = logn - cent_ref[...]
    sc = -(dlt * dlt)
    m = jnp.max(sc, axis=1, keepdims=True)
    ex = jnp.exp(sc - m)
    p_ref[...] = ex / jnp.sum(ex, axis=1, keepdims=True)


def _layer1_body(h_ref, acc_ref, deg_ref, ws_ref, wn_ref, b_ref,
                 o0_ref, o1_ref, o2_ref, o3_ref):
    dg = deg_ref[0, :, 0:1] + deg_ref[1, :, 0:1]
    inv = 1.0 / jnp.maximum(dg, 1.0)
    m1 = (acc_ref[0] + acc_ref[1]) * inv
    h = h_ref[...]
    outs = [o0_ref, o1_ref, o2_ref, o3_ref]
    for e in range(NE):
        ye = (jnp.dot(h, ws_ref[e], preferred_element_type=jnp.float32)
              + jnp.dot(m1, wn_ref[e], preferred_element_type=jnp.float32)
              + b_ref[e:e + 1, :])
        outs[e][...] = jnp.maximum(ye, 0.0)


def _layer2_body(h0_ref, h1_ref, h2_ref, h3_ref, a0_ref, a1_ref, a2_ref,
                 a3_ref, deg_ref, p_ref, ws_ref, wn_ref, b_ref, out_ref):
    dg = deg_ref[0, :, 0:1] + deg_ref[1, :, 0:1]
    inv = 1.0 / jnp.maximum(dg, 1.0)
    p = p_ref[...]
    out = jnp.zeros((RB, D), jnp.float32)
    hes = [h0_ref, h1_ref, h2_ref, h3_ref]
    accs = [a0_ref, a1_ref, a2_ref, a3_ref]
    for e in range(NE):
        m2 = (accs[e][0] + accs[e][1]) * inv
        ye = (jnp.dot(hes[e][...], ws_ref[e], preferred_element_type=jnp.float32)
              + jnp.dot(m2, wn_ref[e], preferred_element_type=jnp.float32)
              + b_ref[e:e + 1, :])
        out = out + p[:, e:e + 1] * ye
    out_ref[...] = out


def kernel(x, edge_index, batch, W_enc, b_enc, Wself1, Wneigh1, b1,
           Wself2, Wneigh2, b2, centers):
    src = edge_index[0].astype(jnp.int32)
    dst = edge_index[1].astype(jnp.int32)
    e_edges = src.shape[0]
    chunks_per_tile = 4 * (-(-e_edges // (TILES * CH * 4)))
    e_pad = chunks_per_tile * TILES * CH
    npad = e_pad - e_edges
    src_p = jnp.concatenate(
        [src, jnp.zeros((npad,), jnp.int32)])
    dst_p = jnp.concatenate(
        [dst, N + (jnp.arange(npad, dtype=jnp.int32) % CH)])
    zrow = jnp.zeros((CH, D), jnp.float32)
    ones128 = jnp.ones((CH, D), jnp.float32)

    batch_i = batch.astype(jnp.int32)
    batch_full = jnp.concatenate(
        [batch_i, jnp.full((NB_PAD - N,), NG, jnp.int32)]).reshape(1, NB_PAD)
    batch_blk = batch_i.reshape(N, 1)

    h, probs = pl.pallas_call(
        _encoder_body,
        grid=(GRID,),
        in_specs=[
            pl.BlockSpec((RB, D), lambda i: (i, 0)),
            pl.BlockSpec((D, D), lambda i: (0, 0)),
            pl.BlockSpec((1, D), lambda i: (0, 0)),
            pl.BlockSpec((1, NB_PAD), lambda i: (0, 0)),
            pl.BlockSpec((RB, 1), lambda i: (i, 0)),
            pl.BlockSpec((1, NE), lambda i: (0, 0)),
        ],
        out_specs=[
            pl.BlockSpec((RB, D), lambda i: (i, 0)),
            pl.BlockSpec((RB, NE), lambda i: (i, 0)),
        ],
        out_shape=[
            jax.ShapeDtypeStruct((N, D), jnp.float32),
            jax.ShapeDtypeStruct((N, NE), jnp.float32),
        ],
    )(x, W_enc, b_enc.reshape(1, D), batch_full, batch_blk,
      centers.reshape(1, NE))

    acc1_f, deg_f = _make_agg_h(chunks_per_tile)(
        h, src_p, dst_p, zrow, ones128)
    acc1 = acc1_f.reshape(NC, N_PAD, D)
    deg = deg_f.reshape(NC, N_PAD, D)

    hes = pl.pallas_call(
        _layer1_body,
        grid=(GRID,),
        in_specs=[
            pl.BlockSpec((RB, D), lambda i: (i, 0)),
            pl.BlockSpec((NC, RB, D), lambda i: (0, i, 0)),
            pl.BlockSpec((NC, RB, D), lambda i: (0, i, 0)),
            pl.BlockSpec((NE, D, D), lambda i: (0, 0, 0)),
            pl.BlockSpec((NE, D, D), lambda i: (0, 0, 0)),
            pl.BlockSpec((NE, D), lambda i: (0, 0)),
        ],
        out_specs=[pl.BlockSpec((RB, D), lambda i: (i, 0))
                   for _ in range(NE)],
        out_shape=[jax.ShapeDtypeStruct((N, D), jnp.float32)
                   for _ in range(NE)],
    )(h, acc1, deg, Wself1, Wneigh1, b1)

    acc2_fs = _make_agg_experts(chunks_per_tile)(
        hes[0], hes[1], hes[2], hes[3], src_p, dst_p, zrow)
    acc2s = [a.reshape(NC, N_PAD, D) for a in acc2_fs]

    out = pl.pallas_call(
        _layer2_body,
        grid=(GRID,),
        in_specs=(
            [pl.BlockSpec((RB, D), lambda i: (i, 0)) for _ in range(NE)]
            + [pl.BlockSpec((NC, RB, D), lambda i: (0, i, 0))
               for _ in range(NE)]
            + [
                pl.BlockSpec((NC, RB, D), lambda i: (0, i, 0)),
                pl.BlockSpec((RB, NE), lambda i: (i, 0)),
                pl.BlockSpec((NE, D, D), lambda i: (0, 0, 0)),
                pl.BlockSpec((NE, D, D), lambda i: (0, 0, 0)),
                pl.BlockSpec((NE, D), lambda i: (0, 0)),
            ]
        ),
        out_specs=pl.BlockSpec((RB, D), lambda i: (i, 0)),
        out_shape=jax.ShapeDtypeStruct((N, D), jnp.float32),
    )(hes[0], hes[1], hes[2], hes[3], acc2s[0], acc2s[1], acc2s[2],
      acc2s[3], deg, probs, Wself2, Wneigh2, b2)
    return out


# speedup vs baseline: 2.2779x; 2.2779x over previous
import functools

import jax
import jax.numpy as jnp
from jax import lax
from jax.experimental import pallas as pl
from jax.experimental.pallas import tpu as pltpu
from jax.experimental.pallas import tpu_sc as plsc

N = 10000
D = 128
NE = 4
NG = 16

NC = 2
NS = 16
TILES = NC * NS
CH = 80
ROWS_PER_TILE = 640
RCH = ROWS_PER_TILE // CH
N_PAD = ROWS_PER_TILE * NS

RB = 1000
GRID = N // RB
NB_PAD = 10240


def _sc_mesh():
    return plsc.VectorSubcoreMesh(core_axis_name="c", subcore_axis_name="s",
                                  num_cores=NC, num_subcores=NS)


def _sc_pipeline(n, T, s_wait, id_issue, id_wait, s_issue,
                 g_issue=None, g_wait=None, is_issue=None, is_wait=None):
    gather = g_issue is not None

    def head_step(k):
        j = k % 4
        id_issue(k, j)
        if gather:
            is_wait(k, j)
            g_issue(k, j)
        if k >= 2:
            jd = (k + 2) % 4
            if gather:
                g_wait(k - 2, jd)
                is_issue(k + 2, jd)
            id_wait(k - 2, jd)
            s_issue(k - 2, jd)
        elif gather:
            is_issue(k + 2, (k + 2) % 4)

    if gather:
        is_issue(0, 0)
        is_issue(1, 1)
    for k in range(4):
        head_step(k)

    def body(t, carry):
        for dlt in range(4):
            k = 4 * t + dlt
            j = dlt
            jd = (dlt + 2) % 4
            s_wait(k - 4, j)
            id_issue(k, j)
            if gather:
                is_wait(k, j)
                g_issue(k, j)
                g_wait(k - 2, jd)
                is_issue(k + 2, jd)
            id_wait(k - 2, jd)
            s_issue(k - 2, jd)
        return carry

    lax.fori_loop(1, T, body, 0)
    s_wait(n - 4, 0)
    if gather:
        g_wait(n - 2, 2)
    id_wait(n - 2, 2)
    s_issue(n - 2, 2)
    s_wait(n - 3, 1)
    if gather:
        g_wait(n - 1, 3)
    id_wait(n - 1, 3)
    s_issue(n - 1, 3)
    if gather:
        is_wait(n, 0)
        is_wait(n + 1, 1)
    s_wait(n - 2, 2)
    s_wait(n - 1, 3)


def _make_ring_ops(base, n, v_hbm, src_hbm, dst_hbm, rbs, sbu, dbu,
                   sgs, sis, sds, sss, acc_sh):
    def is_issue(k, j):
        kk = lax.min(jnp.int32(k), n - 1)
        pltpu.async_copy(src_hbm.at[pl.ds((base + kk) * CH, CH)],
                         sbu[j], sis[j])

    def is_wait(k, j):
        kk = lax.min(jnp.int32(k), n - 1)
        pltpu.make_async_copy(src_hbm.at[pl.ds((base + kk) * CH, CH)],
                              sbu[j], sis[j]).wait()

    def id_issue(k, j):
        pltpu.async_copy(dst_hbm.at[pl.ds((base + k) * CH, CH)],
                         dbu[j], sds[j])

    def id_wait(k, j):
        pltpu.make_async_copy(dst_hbm.at[pl.ds((base + k) * CH, CH)],
                              dbu[j], sds[j]).wait()

    def g_issue(k, j):
        pltpu.async_copy(v_hbm.at[sbu[j]], rbs[j], sgs[j])

    def g_wait(k, j):
        pltpu.make_async_copy(v_hbm.at[sbu[j]], rbs[j], sgs[j]).wait()

    def s_issue(k, j, src_buf=None):
        pltpu.async_copy(rbs[j] if src_buf is None else src_buf,
                         acc_sh.at[dbu[j]], sss[j], add=True)

    def s_wait(k, j, src_buf=None):
        pltpu.make_async_copy(rbs[j] if src_buf is None else src_buf,
                              acc_sh.at[dbu[j]], sss[j]).wait()

    return dict(is_issue=is_issue, is_wait=is_wait, id_issue=id_issue,
                id_wait=id_wait, g_issue=g_issue, g_wait=g_wait,
                s_issue=s_issue, s_wait=s_wait)


_SC_SCRATCH = (
    [pltpu.VMEM((CH, D), jnp.float32) for _ in range(4)]
    + [pltpu.VMEM((CH,), jnp.int32) for _ in range(8)]
    + [pltpu.SemaphoreType.DMA for _ in range(16)]
    + [pltpu.VMEM_SHARED((N_PAD, D), jnp.float32)]
)


def _make_agg_h(n0, n1):
    assert n0 % 4 == 0 and n1 % 4 == 0 and n0 >= 8 and n1 >= 8

    @functools.partial(
        pl.kernel,
        out_type=[
            jax.ShapeDtypeStruct((NC * N_PAD, D), jnp.float32),
            jax.ShapeDtypeStruct((NC * N_PAD, D), jnp.float32),
        ],
        mesh=_sc_mesh(),
        scratch_types=list(_SC_SCRATCH),
    )
    def agg(v_hbm, src_hbm, dst_hbm, zrow_hbm, ones_hbm,
            out_hbm, deg_hbm,
            rb0, rb1, rb2, rb3, sb0, sb1, sb2, sb3, db0, db1, db2, db3,
            sg0, sg1, sg2, sg3, si0, si1, si2, si3,
            sd0, sd1, sd2, sd3, ss0, ss1, ss2, ss3, acc_sh):
        c = lax.axis_index("c")
        s = lax.axis_index("s")
        n = jnp.int32(n0) + c * jnp.int32(n1 - n0)
        T = jnp.int32(n0 // 4) + c * jnp.int32(n1 // 4 - n0 // 4)
        base = s * (n0 + n1) + c * n0
        r0 = s * ROWS_PER_TILE
        o0 = c * N_PAD + r0
        rbs = [rb0, rb1, rb2, rb3]
        ops = _make_ring_ops(base, n, v_hbm, src_hbm, dst_hbm, rbs,
                             [sb0, sb1, sb2, sb3], [db0, db1, db2, db3],
                             [sg0, sg1, sg2, sg3], [si0, si1, si2, si3],
                             [sd0, sd1, sd2, sd3], [ss0, ss1, ss2, ss3],
                             acc_sh)

        def zero_acc():
            pltpu.sync_copy(zrow_hbm, rbs[0])
            for j in range(RCH):
                pltpu.sync_copy(rbs[0], acc_sh.at[pl.ds(r0 + j * CH, CH)])

        def writeout(dst_flat):
            for j in range(RCH):
                pltpu.sync_copy(acc_sh.at[pl.ds(r0 + j * CH, CH)], rbs[0])
                pltpu.sync_copy(rbs[0], dst_flat.at[pl.ds(o0 + j * CH, CH)])

        zero_acc()
        plsc.subcore_barrier()
        _sc_pipeline(n, T, ops["s_wait"], ops["id_issue"], ops["id_wait"],
                     ops["s_issue"], ops["g_issue"], ops["g_wait"],
                     ops["is_issue"], ops["is_wait"])
        plsc.subcore_barrier()
        writeout(out_hbm)

        zero_acc()
        pltpu.sync_copy(ones_hbm, rb1)
        plsc.subcore_barrier()

        def s_issue1(k, j):
            ops["s_issue"](k, j, src_buf=rb1)

        def s_wait1(k, j):
            ops["s_wait"](k, j, src_buf=rb1)

        _sc_pipeline(n, T, s_wait1, ops["id_issue"], ops["id_wait"],
                     s_issue1)
        plsc.subcore_barrier()
        writeout(deg_hbm)

    return agg


def _make_agg_experts(n0, n1):
    assert n0 % 4 == 0 and n1 % 4 == 0 and n0 >= 8 and n1 >= 8

    @functools.partial(
        pl.kernel,
        out_type=[jax.ShapeDtypeStruct((NC * N_PAD, D), jnp.float32)
                  for _ in range(NE)],
        mesh=_sc_mesh(),
        scratch_types=list(_SC_SCRATCH),
    )
    def agg(v0_hbm, v1_hbm, v2_hbm, v3_hbm, src_hbm, dst_hbm, zrow_hbm,
            o0_hbm, o1_hbm, o2_hbm, o3_hbm,
            rb0, rb1, rb2, rb3, sb0, sb1, sb2, sb3, db0, db1, db2, db3,
            sg0, sg1, sg2, sg3, si0, si1, si2, si3,
            sd0, sd1, sd2, sd3, ss0, ss1, ss2, ss3, acc_sh):
        c = lax.axis_index("c")
        s = lax.axis_index("s")
        n = jnp.int32(n0) + c * jnp.int32(n1 - n0)
        T = jnp.int32(n0 // 4) + c * jnp.int32(n1 // 4 - n0 // 4)
        base = s * (n0 + n1) + c * n0
        r0 = s * ROWS_PER_TILE
        o0 = c * N_PAD + r0
        rbs = [rb0, rb1, rb2, rb3]
        vs = [v0_hbm, v1_hbm, v2_hbm, v3_hbm]
        os_ = [o0_hbm, o1_hbm, o2_hbm, o3_hbm]
        for e in range(NE):
            ops = _make_ring_ops(base, n, vs[e], src_hbm, dst_hbm, rbs,
                                 [sb0, sb1, sb2, sb3], [db0, db1, db2, db3],
                                 [sg0, sg1, sg2, sg3], [si0, si1, si2, si3],
                                 [sd0, sd1, sd2, sd3], [ss0, ss1, ss2, ss3],
                                 acc_sh)
            pltpu.sync_copy(zrow_hbm, rbs[0])
            for j in range(RCH):
                pltpu.sync_copy(rbs[0], acc_sh.at[pl.ds(r0 + j * CH, CH)])
            plsc.subcore_barrier()
            _sc_pipeline(n, T, ops["s_wait"], ops["id_issue"],
                         ops["id_wait"], ops["s_issue"], ops["g_issue"],
                         ops["g_wait"], ops["is_issue"], ops["is_wait"])
            plsc.subcore_barrier()
            for j in range(RCH):
                pltpu.sync_copy(acc_sh.at[pl.ds(r0 + j * CH, CH)], rbs[0])
                pltpu.sync_copy(rbs[0], os_[e].at[pl.ds(o0 + j * CH, CH)])

    return agg


def _encoder_body(x_ref, w_ref, b_ref, bfull_ref, bblk_ref, cent_ref,
                  h_ref, p_ref):
    h = jnp.dot(x_ref[...], w_ref[...], preferred_element_type=jnp.float32)
    h_ref[...] = jnp.maximum(h + b_ref[...], 0.0)
    bf = bfull_ref[...]
    bb = bblk_ref[...]
    inv_logn = 1.0 / jnp.log(jnp.float32(N))
    logn = jnp.zeros((RB, 1), jnp.float32)
    for g in range(NG):
        cnt = jnp.sum(jnp.where(bf == g, 1.0, 0.0))
        lg = jnp.log(jnp.maximum(cnt, 1.0)) * inv_logn
        logn = logn + jnp.where(bb == g, lg, 0.0)
    dlt = logn - cent_ref[...]
    sc = -(dlt * dlt)
    m = jnp.max(sc, axis=1, keepdims=True)
    ex = jnp.exp(sc - m)
    p_ref[...] = ex / jnp.sum(ex, axis=1, keepdims=True)


def _layer1_body(h_ref, acc_ref, deg_ref, ws_ref, wn_ref, b_ref,
                 o0_ref, o1_ref, o2_ref, o3_ref):
    dg = deg_ref[0, :, 0:1] + deg_ref[1, :, 0:1]
    inv = 1.0 / jnp.maximum(dg, 1.0)
    m1 = (acc_ref[0] + acc_ref[1]) * inv
    h = h_ref[...]
    outs = [o0_ref, o1_ref, o2_ref, o3_ref]
    for e in range(NE):
        ye = (jnp.dot(h, ws_ref[e], preferred_element_type=jnp.float32)
              + jnp.dot(m1, wn_ref[e], preferred_element_type=jnp.float32)
              + b_ref[e:e + 1, :])
        outs[e][...] = jnp.maximum(ye, 0.0)


def _layer2_body(h0_ref, h1_ref, h2_ref, h3_ref, a0_ref, a1_ref, a2_ref,
                 a3_ref, deg_ref, p_ref, ws_ref, wn_ref, b_ref, out_ref):
    dg = deg_ref[0, :, 0:1] + deg_ref[1, :, 0:1]
    inv = 1.0 / jnp.maximum(dg, 1.0)
    p = p_ref[...]
    out = jnp.zeros((RB, D), jnp.float32)
    hes = [h0_ref, h1_ref, h2_ref, h3_ref]
    accs = [a0_ref, a1_ref, a2_ref, a3_ref]
    for e in range(NE):
        m2 = (accs[e][0] + accs[e][1]) * inv
        ye = (jnp.dot(hes[e][...], ws_ref[e], preferred_element_type=jnp.float32)
              + jnp.dot(m2, wn_ref[e], preferred_element_type=jnp.float32)
              + b_ref[e:e + 1, :])
        out = out + p[:, e:e + 1] * ye
    out_ref[...] = out


def kernel(x, edge_index, batch, W_enc, b_enc, Wself1, Wneigh1, b1,
           Wself2, Wneigh2, b2, centers):
    src = edge_index[0].astype(jnp.int32)
    dst = edge_index[1].astype(jnp.int32)
    e_edges = src.shape[0]
    m_pair = 4 * (-(-e_edges // (NS * CH * 4)))
    n0_c = max(8, 4 * int(round(m_pair * 0.825 / 4.0)))
    n1_c = m_pair - n0_c
    e_pad = m_pair * NS * CH
    npad = e_pad - e_edges
    src_p = jnp.concatenate(
        [src, jnp.zeros((npad,), jnp.int32)])
    dst_p = jnp.concatenate(
        [dst, N + (jnp.arange(npad, dtype=jnp.int32) % CH)])
    zrow = jnp.zeros((CH, D), jnp.float32)
    ones128 = jnp.ones((CH, D), jnp.float32)

    batch_i = batch.astype(jnp.int32)
    batch_full = jnp.concatenate(
        [batch_i, jnp.full((NB_PAD - N,), NG, jnp.int32)]).reshape(1, NB_PAD)
    batch_blk = batch_i.reshape(N, 1)

    h, probs = pl.pallas_call(
        _encoder_body,
        grid=(GRID,),
        in_specs=[
            pl.BlockSpec((RB, D), lambda i: (i, 0)),
            pl.BlockSpec((D, D), lambda i: (0, 0)),
            pl.BlockSpec((1, D), lambda i: (0, 0)),
            pl.BlockSpec((1, NB_PAD), lambda i: (0, 0)),
            pl.BlockSpec((RB, 1), lambda i: (i, 0)),
            pl.BlockSpec((1, NE), lambda i: (0, 0)),
        ],
        out_specs=[
            pl.BlockSpec((RB, D), lambda i: (i, 0)),
            pl.BlockSpec((RB, NE), lambda i: (i, 0)),
        ],
        out_shape=[
            jax.ShapeDtypeStruct((N, D), jnp.float32),
            jax.ShapeDtypeStruct((N, NE), jnp.float32),
        ],
    )(x, W_enc, b_enc.reshape(1, D), batch_full, batch_blk,
      centers.reshape(1, NE))

    acc1_f, deg_f = _make_agg_h(n0_c, n1_c)(
        h, src_p, dst_p, zrow, ones128)
    acc1 = acc1_f.reshape(NC, N_PAD, D)
    deg = deg_f.reshape(NC, N_PAD, D)

    hes = pl.pallas_call(
        _layer1_body,
        grid=(GRID,),
        in_specs=[
            pl.BlockSpec((RB, D), lambda i: (i, 0)),
            pl.BlockSpec((NC, RB, D), lambda i: (0, i, 0)),
            pl.BlockSpec((NC, RB, D), lambda i: (0, i, 0)),
            pl.BlockSpec((NE, D, D), lambda i: (0, 0, 0)),
            pl.BlockSpec((NE, D, D), lambda i: (0, 0, 0)),
            pl.BlockSpec((NE, D), lambda i: (0, 0)),
        ],
        out_specs=[pl.BlockSpec((RB, D), lambda i: (i, 0))
                   for _ in range(NE)],
        out_shape=[jax.ShapeDtypeStruct((N, D), jnp.float32)
                   for _ in range(NE)],
    )(h, acc1, deg, Wself1, Wneigh1, b1)

    acc2_fs = _make_agg_experts(n0_c, n1_c)(
        hes[0], hes[1], hes[2], hes[3], src_p, dst_p, zrow)
    acc2s = [a.reshape(NC, N_PAD, D) for a in acc2_fs]

    out = pl.pallas_call(
        _layer2_body,
        grid=(GRID,),
        in_specs=(
            [pl.BlockSpec((RB, D), lambda i: (i, 0)) for _ in range(NE)]
            + [pl.BlockSpec((NC, RB, D), lambda i: (0, i, 0))
               for _ in range(NE)]
            + [
                pl.BlockSpec((NC, RB, D), lambda i: (0, i, 0)),
                pl.BlockSpec((RB, NE), lambda i: (i, 0)),
                pl.BlockSpec((NE, D, D), lambda i: (0, 0, 0)),
                pl.BlockSpec((NE, D, D), lambda i: (0, 0, 0)),
                pl.BlockSpec((NE, D), lambda i: (0, 0)),
            ]
        ),
        out_specs=pl.BlockSpec((RB, D), lambda i: (i, 0)),
        out_shape=jax.ShapeDtypeStruct((N, D), jnp.float32),
    )(hes[0], hes[1], hes[2], hes[3], acc2s[0], acc2s[1], acc2s[2],
      acc2s[3], deg, probs, Wself2, Wneigh2, b2)
    return out


# speedup vs baseline: 2.2898x; 1.0052x over previous
import functools

import jax
import jax.numpy as jnp
from jax import lax
from jax.experimental import pallas as pl
from jax.experimental.pallas import tpu as pltpu
from jax.experimental.pallas import tpu_sc as plsc

N = 10000
D = 128
NE = 4
NG = 16

NC = 2
NS = 16
TILES = NC * NS
CH = 80
ROWS_PER_TILE = 640
RCH = ROWS_PER_TILE // CH
N_PAD = ROWS_PER_TILE * NS

RB = 1000
GRID = N // RB
NB_PAD = 10240


def _sc_mesh():
    return plsc.VectorSubcoreMesh(core_axis_name="c", subcore_axis_name="s",
                                  num_cores=NC, num_subcores=NS)


def _sc_pipeline(n, T, s_wait, id_issue, id_wait, s_issue,
                 g_issue=None, g_wait=None, is_issue=None, is_wait=None):
    gather = g_issue is not None

    def head_step(k):
        j = k % 4
        id_issue(k, j)
        if gather:
            is_wait(k, j)
            g_issue(k, j)
        if k >= 2:
            jd = (k + 2) % 4
            if gather:
                g_wait(k - 2, jd)
                is_issue(k + 2, jd)
            id_wait(k - 2, jd)
            s_issue(k - 2, jd)
        elif gather:
            is_issue(k + 2, (k + 2) % 4)

    if gather:
        is_issue(0, 0)
        is_issue(1, 1)
    for k in range(4):
        head_step(k)

    def body(t, carry):
        for dlt in range(4):
            k = 4 * t + dlt
            j = dlt
            jd = (dlt + 2) % 4
            s_wait(k - 4, j)
            id_issue(k, j)
            if gather:
                is_wait(k, j)
                g_issue(k, j)
                g_wait(k - 2, jd)
                is_issue(k + 2, jd)
            id_wait(k - 2, jd)
            s_issue(k - 2, jd)
        return carry

    lax.fori_loop(1, T, body, 0)
    s_wait(n - 4, 0)
    if gather:
        g_wait(n - 2, 2)
    id_wait(n - 2, 2)
    s_issue(n - 2, 2)
    s_wait(n - 3, 1)
    if gather:
        g_wait(n - 1, 3)
    id_wait(n - 1, 3)
    s_issue(n - 1, 3)
    if gather:
        is_wait(n, 0)
        is_wait(n + 1, 1)
    s_wait(n - 2, 2)
    s_wait(n - 1, 3)


def _make_ring_ops(base, n, v_hbm, src_hbm, dst_hbm, rbs, sbu, dbu,
                   sgs, sis, sds, sss, acc_sh):
    def is_issue(k, j):
        kk = lax.min(jnp.int32(k), n - 1)
        pltpu.async_copy(src_hbm.at[pl.ds((base + kk) * CH, CH)],
                         sbu[j], sis[j])

    def is_wait(k, j):
        kk = lax.min(jnp.int32(k), n - 1)
        pltpu.make_async_copy(src_hbm.at[pl.ds((base + kk) * CH, CH)],
                              sbu[j], sis[j]).wait()

    def id_issue(k, j):
        pltpu.async_copy(dst_hbm.at[pl.ds((base + k) * CH, CH)],
                         dbu[j], sds[j])

    def id_wait(k, j):
        pltpu.make_async_copy(dst_hbm.at[pl.ds((base + k) * CH, CH)],
                              dbu[j], sds[j]).wait()

    def g_issue(k, j):
        pltpu.async_copy(v_hbm.at[sbu[j]], rbs[j], sgs[j])

    def g_wait(k, j):
        pltpu.make_async_copy(v_hbm.at[sbu[j]], rbs[j], sgs[j]).wait()

    def s_issue(k, j, src_buf=None):
        pltpu.async_copy(rbs[j] if src_buf is None else src_buf,
                         acc_sh.at[dbu[j]], sss[j], add=True)

    def s_wait(k, j, src_buf=None):
        pltpu.make_async_copy(rbs[j] if src_buf is None else src_buf,
                              acc_sh.at[dbu[j]], sss[j]).wait()

    return dict(is_issue=is_issue, is_wait=is_wait, id_issue=id_issue,
                id_wait=id_wait, g_issue=g_issue, g_wait=g_wait,
                s_issue=s_issue, s_wait=s_wait)


_SC_SCRATCH = (
    [pltpu.VMEM((CH, D), jnp.float32) for _ in range(4)]
    + [pltpu.VMEM((CH,), jnp.int32) for _ in range(8)]
    + [pltpu.SemaphoreType.DMA for _ in range(16)]
    + [pltpu.VMEM_SHARED((N_PAD, D), jnp.float32)]
)


def _make_agg_h(m):
    assert m % 4 == 0 and m >= 8

    @functools.partial(
        pl.kernel,
        out_type=[
            jax.ShapeDtypeStruct((N_PAD, D), jnp.float32),
            jax.ShapeDtypeStruct((N_PAD, D), jnp.float32),
        ],
        mesh=_sc_mesh(),
        scratch_types=list(_SC_SCRATCH),
    )
    def agg(v_hbm, src_hbm, dst_hbm, zrow_hbm, ones_hbm,
            out_hbm, deg_hbm,
            rb0, rb1, rb2, rb3, sb0, sb1, sb2, sb3, db0, db1, db2, db3,
            sg0, sg1, sg2, sg3, si0, si1, si2, si3,
            sd0, sd1, sd2, sd3, ss0, ss1, ss2, ss3, acc_sh):
        c = lax.axis_index("c")
        s = lax.axis_index("s")
        base = s * m
        r0 = s * ROWS_PER_TILE
        rbs = [rb0, rb1, rb2, rb3]
        ops = _make_ring_ops(base, jnp.int32(m), v_hbm, src_hbm, dst_hbm,
                             rbs,
                             [sb0, sb1, sb2, sb3], [db0, db1, db2, db3],
                             [sg0, sg1, sg2, sg3], [si0, si1, si2, si3],
                             [sd0, sd1, sd2, sd3], [ss0, ss1, ss2, ss3],
                             acc_sh)
        T = m // 4

        pltpu.sync_copy(zrow_hbm, rbs[0])
        for j in range(RCH):
            pltpu.sync_copy(rbs[0], acc_sh.at[pl.ds(r0 + j * CH, CH)])
        pltpu.sync_copy(ones_hbm, rb1)
        plsc.subcore_barrier()

        @pl.when(c == 0)
        def _gather_phase():
            _sc_pipeline(m, T, ops["s_wait"], ops["id_issue"],
                         ops["id_wait"], ops["s_issue"], ops["g_issue"],
                         ops["g_wait"], ops["is_issue"], ops["is_wait"])

        @pl.when(c == 1)
        def _deg_phase():
            def s_issue1(k, j):
                ops["s_issue"](k, j, src_buf=rb1)

            def s_wait1(k, j):
                ops["s_wait"](k, j, src_buf=rb1)

            _sc_pipeline(m, T, s_wait1, ops["id_issue"], ops["id_wait"],
                         s_issue1)

        plsc.subcore_barrier()
        for j in range(RCH):
            pltpu.sync_copy(acc_sh.at[pl.ds(r0 + j * CH, CH)], rbs[0])

            @pl.when(c == 0)
            def _w_acc(j=j):
                pltpu.sync_copy(rbs[0], out_hbm.at[pl.ds(r0 + j * CH, CH)])

            @pl.when(c == 1)
            def _w_deg(j=j):
                pltpu.sync_copy(rbs[0], deg_hbm.at[pl.ds(r0 + j * CH, CH)])

    return agg


def _make_agg_experts(n0, n1):
    assert n0 % 4 == 0 and n1 % 4 == 0 and n0 >= 8 and n1 >= 8

    @functools.partial(
        pl.kernel,
        out_type=[jax.ShapeDtypeStruct((NC * N_PAD, D), jnp.float32)
                  for _ in range(NE)],
        mesh=_sc_mesh(),
        scratch_types=list(_SC_SCRATCH),
    )
    def agg(v0_hbm, v1_hbm, v2_hbm, v3_hbm, src_hbm, dst_hbm, zrow_hbm,
            o0_hbm, o1_hbm, o2_hbm, o3_hbm,
            rb0, rb1, rb2, rb3, sb0, sb1, sb2, sb3, db0, db1, db2, db3,
            sg0, sg1, sg2, sg3, si0, si1, si2, si3,
            sd0, sd1, sd2, sd3, ss0, ss1, ss2, ss3, acc_sh):
        c = lax.axis_index("c")
        s = lax.axis_index("s")
        n = jnp.int32(n0) + c * jnp.int32(n1 - n0)
        T = jnp.int32(n0 // 4) + c * jnp.int32(n1 // 4 - n0 // 4)
        base = s * (n0 + n1) + c * n0
        r0 = s * ROWS_PER_TILE
        o0 = c * N_PAD + r0
        rbs = [rb0, rb1, rb2, rb3]
        vs = [v0_hbm, v1_hbm, v2_hbm, v3_hbm]
        os_ = [o0_hbm, o1_hbm, o2_hbm, o3_hbm]
        for e in range(NE):
            ops = _make_ring_ops(base, n, vs[e], src_hbm, dst_hbm, rbs,
                                 [sb0, sb1, sb2, sb3], [db0, db1, db2, db3],
                                 [sg0, sg1, sg2, sg3], [si0, si1, si2, si3],
                                 [sd0, sd1, sd2, sd3], [ss0, ss1, ss2, ss3],
                                 acc_sh)
            pltpu.sync_copy(zrow_hbm, rbs[0])
            for j in range(RCH):
                pltpu.sync_copy(rbs[0], acc_sh.at[pl.ds(r0 + j * CH, CH)])
            plsc.subcore_barrier()
            _sc_pipeline(n, T, ops["s_wait"], ops["id_issue"],
                         ops["id_wait"], ops["s_issue"], ops["g_issue"],
                         ops["g_wait"], ops["is_issue"], ops["is_wait"])
            plsc.subcore_barrier()
            for j in range(RCH):
                pltpu.sync_copy(acc_sh.at[pl.ds(r0 + j * CH, CH)], rbs[0])
                pltpu.sync_copy(rbs[0], os_[e].at[pl.ds(o0 + j * CH, CH)])

    return agg


def _encoder_body(x_ref, w_ref, b_ref, bfull_ref, bblk_ref, cent_ref,
                  h_ref, p_ref):
    h = jnp.dot(x_ref[...], w_ref[...], preferred_element_type=jnp.float32)
    h_ref[...] = jnp.maximum(h + b_ref[...], 0.0)
    bf = bfull_ref[...]
    bb = bblk_ref[...]
    inv_logn = 1.0 / jnp.log(jnp.float32(N))
    logn = jnp.zeros((RB, 1), jnp.float32)
    for g in range(NG):
        cnt = jnp.sum(jnp.where(bf == g, 1.0, 0.0))
        lg = jnp.log(jnp.maximum(cnt, 1.0)) * inv_logn
        logn = logn + jnp.where(bb == g, lg, 0.0)
    dlt = logn - cent_ref[...]
    sc = -(dlt * dlt)
    m = jnp.max(sc, axis=1, keepdims=True)
    ex = jnp.exp(sc - m)
    p_ref[...] = ex / jnp.sum(ex, axis=1, keepdims=True)


def _layer1_body(h_ref, acc_ref, deg_ref, ws_ref, wn_ref, b_ref,
                 o0_ref, o1_ref, o2_ref, o3_ref):
    dg = deg_ref[:, 0:1]
    inv = 1.0 / jnp.maximum(dg, 1.0)
    m1 = acc_ref[...] * inv
    h = h_ref[...]
    outs = [o0_ref, o1_ref, o2_ref, o3_ref]
    for e in range(NE):
        ye = (jnp.dot(h, ws_ref[e], preferred_element_type=jnp.float32)
              + jnp.dot(m1, wn_ref[e], preferred_element_type=jnp.float32)
              + b_ref[e:e + 1, :])
        outs[e][...] = jnp.maximum(ye, 0.0)


def _layer2_body(h0_ref, h1_ref, h2_ref, h3_ref, a0_ref, a1_ref, a2_ref,
                 a3_ref, deg_ref, p_ref, ws_ref, wn_ref, b_ref, out_ref):
    dg = deg_ref[:, 0:1]
    inv = 1.0 / jnp.maximum(dg, 1.0)
    p = p_ref[...]
    out = jnp.zeros((RB, D), jnp.float32)
    hes = [h0_ref, h1_ref, h2_ref, h3_ref]
    accs = [a0_ref, a1_ref, a2_ref, a3_ref]
    for e in range(NE):
        m2 = (accs[e][0] + accs[e][1]) * inv
        ye = (jnp.dot(hes[e][...], ws_ref[e], preferred_element_type=jnp.float32)
              + jnp.dot(m2, wn_ref[e], preferred_element_type=jnp.float32)
              + b_ref[e:e + 1, :])
        out = out + p[:, e:e + 1] * ye
    out_ref[...] = out


def kernel(x, edge_index, batch, W_enc, b_enc, Wself1, Wneigh1, b1,
           Wself2, Wneigh2, b2, centers):
    src = edge_index[0].astype(jnp.int32)
    dst = edge_index[1].astype(jnp.int32)
    e_edges = src.shape[0]
    m_pair = 4 * (-(-e_edges // (NS * CH * 4)))
    n0_c = max(8, 4 * int(round(m_pair * 0.825 / 4.0)))
    n1_c = m_pair - n0_c
    e_pad = m_pair * NS * CH
    npad = e_pad - e_edges
    src_p = jnp.concatenate(
        [src, jnp.zeros((npad,), jnp.int32)])
    dst_p = jnp.concatenate(
        [dst, N + (jnp.arange(npad, dtype=jnp.int32) % CH)])
    zrow = jnp.zeros((CH, D), jnp.float32)
    ones128 = jnp.ones((CH, D), jnp.float32)

    batch_i = batch.astype(jnp.int32)
    batch_full = jnp.concatenate(
        [batch_i, jnp.full((NB_PAD - N,), NG, jnp.int32)]).reshape(1, NB_PAD)
    batch_blk = batch_i.reshape(N, 1)

    h, probs = pl.pallas_call(
        _encoder_body,
        grid=(GRID,),
        in_specs=[
            pl.BlockSpec((RB, D), lambda i: (i, 0)),
            pl.BlockSpec((D, D), lambda i: (0, 0)),
            pl.BlockSpec((1, D), lambda i: (0, 0)),
            pl.BlockSpec((1, NB_PAD), lambda i: (0, 0)),
            pl.BlockSpec((RB, 1), lambda i: (i, 0)),
            pl.BlockSpec((1, NE), lambda i: (0, 0)),
        ],
        out_specs=[
            pl.BlockSpec((RB, D), lambda i: (i, 0)),
            pl.BlockSpec((RB, NE), lambda i: (i, 0)),
        ],
        out_shape=[
            jax.ShapeDtypeStruct((N, D), jnp.float32),
            jax.ShapeDtypeStruct((N, NE), jnp.float32),
        ],
    )(x, W_enc, b_enc.reshape(1, D), batch_full, batch_blk,
      centers.reshape(1, NE))

    acc1, deg = _make_agg_h(m_pair)(h, src_p, dst_p, zrow, ones128)

    hes = pl.pallas_call(
        _layer1_body,
        grid=(GRID,),
        in_specs=[
            pl.BlockSpec((RB, D), lambda i: (i, 0)),
            pl.BlockSpec((RB, D), lambda i: (i, 0)),
            pl.BlockSpec((RB, D), lambda i: (i, 0)),
            pl.BlockSpec((NE, D, D), lambda i: (0, 0, 0)),
            pl.BlockSpec((NE, D, D), lambda i: (0, 0, 0)),
            pl.BlockSpec((NE, D), lambda i: (0, 0)),
        ],
        out_specs=[pl.BlockSpec((RB, D), lambda i: (i, 0))
                   for _ in range(NE)],
        out_shape=[jax.ShapeDtypeStruct((N, D), jnp.float32)
                   for _ in range(NE)],
    )(h, acc1, deg, Wself1, Wneigh1, b1)

    acc2_fs = _make_agg_experts(n0_c, n1_c)(
        hes[0], hes[1], hes[2], hes[3], src_p, dst_p, zrow)
    acc2s = [a.reshape(NC, N_PAD, D) for a in acc2_fs]

    out = pl.pallas_call(
        _layer2_body,
        grid=(GRID,),
        in_specs=(
            [pl.BlockSpec((RB, D), lambda i: (i, 0)) for _ in range(NE)]
            + [pl.BlockSpec((NC, RB, D), lambda i: (0, i, 0))
               for _ in range(NE)]
            + [
                pl.BlockSpec((RB, D), lambda i: (i, 0)),
                pl.BlockSpec((RB, NE), lambda i: (i, 0)),
                pl.BlockSpec((NE, D, D), lambda i: (0, 0, 0)),
                pl.BlockSpec((NE, D, D), lambda i: (0, 0, 0)),
                pl.BlockSpec((NE, D), lambda i: (0, 0)),
            ]
        ),
        out_specs=pl.BlockSpec((RB, D), lambda i: (i, 0)),
        out_shape=jax.ShapeDtypeStruct((N, D), jnp.float32),
    )(hes[0], hes[1], hes[2], hes[3], acc2s[0], acc2s[1], acc2s[2],
      acc2s[3], deg, probs, Wself2, Wneigh2, b2)
    return out


# speedup vs baseline: 2.2898x; 1.0000x over previous
import functools

import jax
import jax.numpy as jnp
from jax import lax
from jax.experimental import pallas as pl
from jax.experimental.pallas import tpu as pltpu
from jax.experimental.pallas import tpu_sc as plsc

N = 10000
D = 128
NE = 4
NG = 16

NC = 2
NS = 16
TILES = NC * NS
CH = 80
ROWS_PER_TILE = 640
RCH = ROWS_PER_TILE // CH
N_PAD = ROWS_PER_TILE * NS

RB = 1000
GRID = N // RB
NB_PAD = 10240


def _sc_mesh():
    return plsc.VectorSubcoreMesh(core_axis_name="c", subcore_axis_name="s",
                                  num_cores=NC, num_subcores=NS)


def _sc_pipeline(n, T, s_wait, id_issue, id_wait, s_issue,
                 g_issue=None, g_wait=None, is_issue=None, is_wait=None):
    gather = g_issue is not None

    def head_step(k):
        j = k % 4
        id_issue(k, j)
        if gather:
            is_wait(k, j)
            g_issue(k, j)
        if k >= 2:
            jd = (k + 2) % 4
            if gather:
                g_wait(k - 2, jd)
                is_issue(k + 2, jd)
            id_wait(k - 2, jd)
            s_issue(k - 2, jd)
        elif gather:
            is_issue(k + 2, (k + 2) % 4)

    if gather:
        is_issue(0, 0)
        is_issue(1, 1)
    for k in range(4):
        head_step(k)

    def body(t, carry):
        for dlt in range(4):
            k = 4 * t + dlt
            j = dlt
            jd = (dlt + 2) % 4
            s_wait(k - 4, j)
            id_issue(k, j)
            if gather:
                is_wait(k, j)
                g_issue(k, j)
                g_wait(k - 2, jd)
                is_issue(k + 2, jd)
            id_wait(k - 2, jd)
            s_issue(k - 2, jd)
        return carry

    lax.fori_loop(1, T, body, 0)
    s_wait(n - 4, 0)
    if gather:
        g_wait(n - 2, 2)
    id_wait(n - 2, 2)
    s_issue(n - 2, 2)
    s_wait(n - 3, 1)
    if gather:
        g_wait(n - 1, 3)
    id_wait(n - 1, 3)
    s_issue(n - 1, 3)
    if gather:
        is_wait(n, 0)
        is_wait(n + 1, 1)
    s_wait(n - 2, 2)
    s_wait(n - 1, 3)


def _make_ring_ops(base, n, v_hbm, src_hbm, dst_hbm, rbs, sbu, dbu,
                   sgs, sis, sds, sss, acc_sh):
    def is_issue(k, j):
        kk = lax.min(jnp.int32(k), n - 1)
        pltpu.async_copy(src_hbm.at[pl.ds((base + kk) * CH, CH)],
                         sbu[j], sis[j])

    def is_wait(k, j):
        kk = lax.min(jnp.int32(k), n - 1)
        pltpu.make_async_copy(src_hbm.at[pl.ds((base + kk) * CH, CH)],
                              sbu[j], sis[j]).wait()

    def id_issue(k, j):
        pltpu.async_copy(dst_hbm.at[pl.ds((base + k) * CH, CH)],
                         dbu[j], sds[j])

    def id_wait(k, j):
        pltpu.make_async_copy(dst_hbm.at[pl.ds((base + k) * CH, CH)],
                              dbu[j], sds[j]).wait()

    def g_issue(k, j):
        pltpu.async_copy(v_hbm.at[sbu[j]], rbs[j], sgs[j])

    def g_wait(k, j):
        pltpu.make_async_copy(v_hbm.at[sbu[j]], rbs[j], sgs[j]).wait()

    def s_issue(k, j, src_buf=None):
        pltpu.async_copy(rbs[j] if src_buf is None else src_buf,
                         acc_sh.at[dbu[j]], sss[j], add=True)

    def s_wait(k, j, src_buf=None):
        pltpu.make_async_copy(rbs[j] if src_buf is None else src_buf,
                              acc_sh.at[dbu[j]], sss[j]).wait()

    return dict(is_issue=is_issue, is_wait=is_wait, id_issue=id_issue,
                id_wait=id_wait, g_issue=g_issue, g_wait=g_wait,
                s_issue=s_issue, s_wait=s_wait)


def _sc_scratch():
    return (
        [pltpu.VMEM((CH, D), jnp.float32) for _ in range(4)]
        + [pltpu.VMEM((CH,), jnp.int32) for _ in range(8)]
        + [pltpu.SemaphoreType.DMA for _ in range(16)]
        + [pltpu.VMEM_SHARED((N_PAD, D), jnp.float32)]
    )


def _make_agg_h(m):
    assert m % 4 == 0 and m >= 8

    @functools.partial(
        pl.kernel,
        out_type=[
            jax.ShapeDtypeStruct((N_PAD, D), jnp.float32),
            jax.ShapeDtypeStruct((N_PAD, D), jnp.float32),
        ],
        mesh=_sc_mesh(),
        scratch_types=_sc_scratch(),
    )
    def agg(v_hbm, src_hbm, dst_hbm, zrow_hbm, ones_hbm,
            out_hbm, deg_hbm,
            rb0, rb1, rb2, rb3, sb0, sb1, sb2, sb3, db0, db1, db2, db3,
            sg0, sg1, sg2, sg3, si0, si1, si2, si3,
            sd0, sd1, sd2, sd3, ss0, ss1, ss2, ss3, acc_sh):
        c = lax.axis_index("c")
        s = lax.axis_index("s")
        base = s * m
        r0 = s * ROWS_PER_TILE
        rbs = [rb0, rb1, rb2, rb3]
        ops = _make_ring_ops(base, jnp.int32(m), v_hbm, src_hbm, dst_hbm,
                             rbs,
                             [sb0, sb1, sb2, sb3], [db0, db1, db2, db3],
                             [sg0, sg1, sg2, sg3], [si0, si1, si2, si3],
                             [sd0, sd1, sd2, sd3], [ss0, ss1, ss2, ss3],
                             acc_sh)
        T = m // 4

        pltpu.sync_copy(zrow_hbm, rbs[0])
        for j in range(RCH):
            pltpu.sync_copy(rbs[0], acc_sh.at[pl.ds(r0 + j * CH, CH)])
        pltpu.sync_copy(ones_hbm, rb1)
        plsc.subcore_barrier()

        @pl.when(c == 0)
        def _gather_phase():
            _sc_pipeline(m, T, ops["s_wait"], ops["id_issue"],
                         ops["id_wait"], ops["s_issue"], ops["g_issue"],
                         ops["g_wait"], ops["is_issue"], ops["is_wait"])

        @pl.when(c == 1)
        def _deg_phase():
            def s_issue1(k, j):
                ops["s_issue"](k, j, src_buf=rb1)

            def s_wait1(k, j):
                ops["s_wait"](k, j, src_buf=rb1)

            _sc_pipeline(m, T, s_wait1, ops["id_issue"], ops["id_wait"],
                         s_issue1)

        plsc.subcore_barrier()
        for j in range(RCH):
            pltpu.sync_copy(acc_sh.at[pl.ds(r0 + j * CH, CH)], rbs[0])

            @pl.when(c == 0)
            def _w_acc(j=j):
                pltpu.sync_copy(rbs[0], out_hbm.at[pl.ds(r0 + j * CH, CH)])

            @pl.when(c == 1)
            def _w_deg(j=j):
                pltpu.sync_copy(rbs[0], deg_hbm.at[pl.ds(r0 + j * CH, CH)])

    return agg


def _make_agg_experts(n0, n1):
    assert n0 % 4 == 0 and n1 % 4 == 0 and n0 >= 8 and n1 >= 8

    @functools.partial(
        pl.kernel,
        out_type=[jax.ShapeDtypeStruct((NC * N_PAD, D), jnp.float32)
                  for _ in range(NE)],
        mesh=_sc_mesh(),
        scratch_types=_sc_scratch(),
    )
    def agg(v0_hbm, v1_hbm, v2_hbm, v3_hbm, src_hbm, dst_hbm, zrow_hbm,
            o0_hbm, o1_hbm, o2_hbm, o3_hbm,
            rb0, rb1, rb2, rb3, sb0, sb1, sb2, sb3, db0, db1, db2, db3,
            sg0, sg1, sg2, sg3, si0, si1, si2, si3,
            sd0, sd1, sd2, sd3, ss0, ss1, ss2, ss3, acc_sh):
        c = lax.axis_index("c")
        s = lax.axis_index("s")
        n = jnp.int32(n0) + c * jnp.int32(n1 - n0)
        T = jnp.int32(n0 // 4) + c * jnp.int32(n1 // 4 - n0 // 4)
        base = s * (n0 + n1) + c * n0
        r0 = s * ROWS_PER_TILE
        o0 = c * N_PAD + r0
        rbs = [rb0, rb1, rb2, rb3]
        vs = [v0_hbm, v1_hbm, v2_hbm, v3_hbm]
        os_ = [o0_hbm, o1_hbm, o2_hbm, o3_hbm]
        for e in range(NE):
            ops = _make_ring_ops(base, n, vs[e], src_hbm, dst_hbm, rbs,
                                 [sb0, sb1, sb2, sb3], [db0, db1, db2, db3],
                                 [sg0, sg1, sg2, sg3], [si0, si1, si2, si3],
                                 [sd0, sd1, sd2, sd3], [ss0, ss1, ss2, ss3],
                                 acc_sh)
            pltpu.sync_copy(zrow_hbm, rbs[0])
            for j in range(RCH):
                pltpu.sync_copy(rbs[0], acc_sh.at[pl.ds(r0 + j * CH, CH)])
            plsc.subcore_barrier()
            _sc_pipeline(n, T, ops["s_wait"], ops["id_issue"],
                         ops["id_wait"], ops["s_issue"], ops["g_issue"],
                         ops["g_wait"], ops["is_issue"], ops["is_wait"])
            plsc.subcore_barrier()
            for j in range(RCH):
                pltpu.sync_copy(acc_sh.at[pl.ds(r0 + j * CH, CH)], rbs[0])
                pltpu.sync_copy(rbs[0], os_[e].at[pl.ds(o0 + j * CH, CH)])

    return agg


def _encoder_body(x_ref, w_ref, b_ref, bfull_ref, bblk_ref, cent_ref,
                  h_ref, p_ref):
    h = jnp.dot(x_ref[...], w_ref[...], preferred_element_type=jnp.float32)
    h_ref[...] = jnp.maximum(h + b_ref[...], 0.0)
    bf = bfull_ref[...]
    bb = bblk_ref[...]
    inv_logn = 1.0 / jnp.log(jnp.float32(N))
    logn = jnp.zeros((RB, 1), jnp.float32)
    for g in range(NG):
        cnt = jnp.sum(jnp.where(bf == g, 1.0, 0.0))
        lg = jnp.log(jnp.maximum(cnt, 1.0)) * inv_logn
        logn = logn + jnp.where(bb == g, lg, 0.0)
    dlt = logn - cent_ref[...]
    sc = -(dlt * dlt)
    m = jnp.max(sc, axis=1, keepdims=True)
    ex = jnp.exp(sc - m)
    p_ref[...] = ex / jnp.sum(ex, axis=1, keepdims=True)


def _layer1_body(h_ref, acc_ref, deg_ref, ws_ref, wn_ref, b_ref,
                 o0_ref, o1_ref, o2_ref, o3_ref):
    dg = deg_ref[:, 0:1]
    inv = 1.0 / jnp.maximum(dg, 1.0)
    m1 = acc_ref[...] * inv
    h = h_ref[...]
    outs = [o0_ref, o1_ref, o2_ref, o3_ref]
    for e in range(NE):
        ye = (jnp.dot(h, ws_ref[e], preferred_element_type=jnp.float32)
              + jnp.dot(m1, wn_ref[e], preferred_element_type=jnp.float32)
              + b_ref[e:e + 1, :])
        outs[e][...] = jnp.maximum(ye, 0.0)


def _layer2_body(h0_ref, h1_ref, h2_ref, h3_ref, a0_ref, a1_ref, a2_ref,
                 a3_ref, deg_ref, p_ref, ws_ref, wn_ref, b_ref, out_ref):
    dg = deg_ref[:, 0:1]
    inv = 1.0 / jnp.maximum(dg, 1.0)
    p = p_ref[...]
    out = jnp.zeros((RB, D), jnp.float32)
    hes = [h0_ref, h1_ref, h2_ref, h3_ref]
    accs = [a0_ref, a1_ref, a2_ref, a3_ref]
    for e in range(NE):
        m2 = (accs[e][0] + accs[e][1]) * inv
        ye = (jnp.dot(hes[e][...], ws_ref[e], preferred_element_type=jnp.float32)
              + jnp.dot(m2, wn_ref[e], preferred_element_type=jnp.float32)
              + b_ref[e:e + 1, :])
        out = out + p[:, e:e + 1] * ye
    out_ref[...] = out


def kernel(x, edge_index, batch, W_enc, b_enc, Wself1, Wneigh1, b1,
           Wself2, Wneigh2, b2, centers):
    src = edge_index[0].astype(jnp.int32)
    dst = edge_index[1].astype(jnp.int32)
    e_edges = src.shape[0]
    m_pair = 4 * (-(-e_edges // (NS * CH * 4)))
    n0_c = max(8, 4 * int(round(m_pair * 0.825 / 4.0)))
    n1_c = m_pair - n0_c
    e_pad = m_pair * NS * CH
    npad = e_pad - e_edges
    src_p = jnp.concatenate(
        [src, jnp.zeros((npad,), jnp.int32)])
    dst_p = jnp.concatenate(
        [dst, N + (jnp.arange(npad, dtype=jnp.int32) % CH)])
    zrow = jnp.zeros((CH, D), jnp.float32)
    ones128 = jnp.ones((CH, D), jnp.float32)

    batch_i = batch.astype(jnp.int32)
    batch_full = jnp.concatenate(
        [batch_i, jnp.full((NB_PAD - N,), NG, jnp.int32)]).reshape(1, NB_PAD)
    batch_blk = batch_i.reshape(N, 1)

    h, probs = pl.pallas_call(
        _encoder_body,
        grid=(GRID,),
        in_specs=[
            pl.BlockSpec((RB, D), lambda i: (i, 0)),
            pl.BlockSpec((D, D), lambda i: (0, 0)),
            pl.BlockSpec((1, D), lambda i: (0, 0)),
            pl.BlockSpec((1, NB_PAD), lambda i: (0, 0)),
            pl.BlockSpec((RB, 1), lambda i: (i, 0)),
            pl.BlockSpec((1, NE), lambda i: (0, 0)),
        ],
        out_specs=[
            pl.BlockSpec((RB, D), lambda i: (i, 0)),
            pl.BlockSpec((RB, NE), lambda i: (i, 0)),
        ],
        out_shape=[
            jax.ShapeDtypeStruct((N, D), jnp.float32),
            jax.ShapeDtypeStruct((N, NE), jnp.float32),
        ],
    )(x, W_enc, b_enc.reshape(1, D), batch_full, batch_blk,
      centers.reshape(1, NE))

    acc1, deg = _make_agg_h(m_pair)(h, src_p, dst_p, zrow, ones128)

    hes = pl.pallas_call(
        _layer1_body,
        grid=(GRID,),
        in_specs=[
            pl.BlockSpec((RB, D), lambda i: (i, 0)),
            pl.BlockSpec((RB, D), lambda i: (i, 0)),
            pl.BlockSpec((RB, D), lambda i: (i, 0)),
            pl.BlockSpec((NE, D, D), lambda i: (0, 0, 0)),
            pl.BlockSpec((NE, D, D), lambda i: (0, 0, 0)),
            pl.BlockSpec((NE, D), lambda i: (0, 0)),
        ],
        out_specs=[pl.BlockSpec((RB, D), lambda i: (i, 0))
                   for _ in range(NE)],
        out_shape=[jax.ShapeDtypeStruct((N, D), jnp.float32)
                   for _ in range(NE)],
    )(h, acc1, deg, Wself1, Wneigh1, b1)

    acc2_fs = _make_agg_experts(n0_c, n1_c)(
        hes[0], hes[1], hes[2], hes[3], src_p, dst_p, zrow)
    acc2s = [a.reshape(NC, N_PAD, D) for a in acc2_fs]

    out = pl.pallas_call(
        _layer2_body,
        grid=(GRID,),
        in_specs=(
            [pl.BlockSpec((RB, D), lambda i: (i, 0)) for _ in range(NE)]
            + [pl.BlockSpec((NC, RB, D), lambda i: (0, i, 0))
               for _ in range(NE)]
            + [
                pl.BlockSpec((RB, D), lambda i: (i, 0)),
                pl.BlockSpec((RB, NE), lambda i: (i, 0)),
                pl.BlockSpec((NE, D, D), lambda i: (0, 0, 0)),
                pl.BlockSpec((NE, D, D), lambda i: (0, 0, 0)),
                pl.BlockSpec((NE, D), lambda i: (0, 0)),
            ]
        ),
        out_specs=pl.BlockSpec((RB, D), lambda i: (i, 0)),
        out_shape=jax.ShapeDtypeStruct((N, D), jnp.float32),
    )(hes[0], hes[1], hes[2], hes[3], acc2s[0], acc2s[1], acc2s[2],
      acc2s[3], deg, probs, Wself2, Wneigh2, b2)
    return out


# speedup vs baseline: 2.3099x; 1.0088x over previous
import functools

import jax
import jax.numpy as jnp
from jax import lax
from jax.experimental import pallas as pl
from jax.experimental.pallas import tpu as pltpu
from jax.experimental.pallas import tpu_sc as plsc

N = 10000
D = 128
NE = 4
NG = 16

NC = 2
NS = 16
TILES = NC * NS
CH = 160
ROWS_PER_TILE = 640
RCH = ROWS_PER_TILE // CH
N_PAD = ROWS_PER_TILE * NS

RB = 1000
GRID = N // RB
NB_PAD = 10240


def _sc_mesh():
    return plsc.VectorSubcoreMesh(core_axis_name="c", subcore_axis_name="s",
                                  num_cores=NC, num_subcores=NS)


def _sc_pipeline(n, T, s_wait, id_issue, id_wait, s_issue,
                 g_issue=None, g_wait=None, is_issue=None, is_wait=None,
                 R=4, d=2):
    gather = g_issue is not None

    def head_step(k):
        j = k % R
        id_issue(k, j)
        if gather:
            is_wait(k, j)
            g_issue(k, j)
        if k >= d:
            jd = (k - d) % R
            if gather:
                g_wait(k - d, jd)
                is_issue(k + d, jd)
            id_wait(k - d, jd)
            s_issue(k - d, jd)
        elif gather:
            is_issue(k + d, (k + d) % R)

    if gather:
        for i in range(d):
            is_issue(i, i % R)
    for k in range(R):
        head_step(k)

    def body(t, carry):
        for dlt in range(R):
            k = R * t + dlt
            j = dlt
            jd = (dlt - d) % R
            s_wait(k - R, j)
            id_issue(k, j)
            if gather:
                is_wait(k, j)
                g_issue(k, j)
                g_wait(k - d, jd)
                is_issue(k + d, jd)
            id_wait(k - d, jd)
            s_issue(k - d, jd)
        return carry

    lax.fori_loop(1, T, body, 0)
    for i in range(d):
        k = n + i
        s_wait(k - R, i % R)
        jd = (i - d) % R
        if gather:
            g_wait(k - d, jd)
        id_wait(k - d, jd)
        s_issue(k - d, jd)
    if gather:
        for i in range(d):
            is_wait(n + i, i % R)
    for i in range(d):
        s_wait(n - d + i, (R - d + i) % R)


def _make_ring_ops(base, n, v_hbm, src_hbm, dst_hbm, rbs, sbu, dbu,
                   sgs, sis, sds, sss, acc_sh):
    def is_issue(k, j):
        kk = lax.min(jnp.int32(k), n - 1)
        pltpu.async_copy(src_hbm.at[pl.ds((base + kk) * CH, CH)],
                         sbu[j], sis[j])

    def is_wait(k, j):
        kk = lax.min(jnp.int32(k), n - 1)
        pltpu.make_async_copy(src_hbm.at[pl.ds((base + kk) * CH, CH)],
                              sbu[j], sis[j]).wait()

    def id_issue(k, j):
        pltpu.async_copy(dst_hbm.at[pl.ds((base + k) * CH, CH)],
                         dbu[j], sds[j])

    def id_wait(k, j):
        pltpu.make_async_copy(dst_hbm.at[pl.ds((base + k) * CH, CH)],
                              dbu[j], sds[j]).wait()

    def g_issue(k, j):
        pltpu.async_copy(v_hbm.at[sbu[j]], rbs[j], sgs[j])

    def g_wait(k, j):
        pltpu.make_async_copy(v_hbm.at[sbu[j]], rbs[j], sgs[j]).wait()

    def s_issue(k, j, src_buf=None):
        pltpu.async_copy(rbs[j] if src_buf is None else src_buf,
                         acc_sh.at[dbu[j]], sss[j], add=True)

    def s_wait(k, j, src_buf=None):
        pltpu.make_async_copy(rbs[j] if src_buf is None else src_buf,
                              acc_sh.at[dbu[j]], sss[j]).wait()

    return dict(is_issue=is_issue, is_wait=is_wait, id_issue=id_issue,
                id_wait=id_wait, g_issue=g_issue, g_wait=g_wait,
                s_issue=s_issue, s_wait=s_wait)


def _sc_scratch():
    return (
        [pltpu.VMEM((CH, D), jnp.float32) for _ in range(2)]
        + [pltpu.VMEM((CH,), jnp.int32) for _ in range(4)]
        + [pltpu.SemaphoreType.DMA for _ in range(8)]
        + [pltpu.VMEM_SHARED((N_PAD, D), jnp.float32)]
    )


def _make_agg_h(m):
    assert m % 2 == 0 and m >= 8

    @functools.partial(
        pl.kernel,
        out_type=[
            jax.ShapeDtypeStruct((N_PAD, D), jnp.float32),
            jax.ShapeDtypeStruct((N_PAD, D), jnp.float32),
        ],
        mesh=_sc_mesh(),
        scratch_types=_sc_scratch(),
    )
    def agg(v_hbm, src_hbm, dst_hbm, zrow_hbm, ones_hbm,
            out_hbm, deg_hbm,
            rb0, rb1, sb0, sb1, db0, db1,
            sg0, sg1, si0, si1, sd0, sd1, ss0, ss1, acc_sh):
        c = lax.axis_index("c")
        s = lax.axis_index("s")
        base = s * m
        r0 = s * ROWS_PER_TILE
        rbs = [rb0, rb1]
        ops = _make_ring_ops(base, jnp.int32(m), v_hbm, src_hbm, dst_hbm,
                             rbs,
                             [sb0, sb1], [db0, db1],
                             [sg0, sg1], [si0, si1],
                             [sd0, sd1], [ss0, ss1],
                             acc_sh)
        T = m // 2

        pltpu.sync_copy(zrow_hbm, rbs[0])
        for j in range(RCH):
            pltpu.sync_copy(rbs[0], acc_sh.at[pl.ds(r0 + j * CH, CH)])
        pltpu.sync_copy(ones_hbm, rb1)
        plsc.subcore_barrier()

        @pl.when(c == 0)
        def _gather_phase():
            _sc_pipeline(m, T, ops["s_wait"], ops["id_issue"],
                         ops["id_wait"], ops["s_issue"], ops["g_issue"],
                         ops["g_wait"], ops["is_issue"], ops["is_wait"],
                         R=2, d=1)

        @pl.when(c == 1)
        def _deg_phase():
            def s_issue1(k, j):
                ops["s_issue"](k, j, src_buf=rb1)

            def s_wait1(k, j):
                ops["s_wait"](k, j, src_buf=rb1)

            _sc_pipeline(m, T, s_wait1, ops["id_issue"], ops["id_wait"],
                         s_issue1, R=2, d=1)

        plsc.subcore_barrier()
        for j in range(RCH):
            pltpu.sync_copy(acc_sh.at[pl.ds(r0 + j * CH, CH)], rbs[0])

            @pl.when(c == 0)
            def _w_acc(j=j):
                pltpu.sync_copy(rbs[0], out_hbm.at[pl.ds(r0 + j * CH, CH)])

            @pl.when(c == 1)
            def _w_deg(j=j):
                pltpu.sync_copy(rbs[0], deg_hbm.at[pl.ds(r0 + j * CH, CH)])

    return agg


def _make_agg_experts(n0, n1):
    assert n0 % 2 == 0 and n1 % 2 == 0 and n0 >= 8 and n1 >= 8

    @functools.partial(
        pl.kernel,
        out_type=[jax.ShapeDtypeStruct((NC * N_PAD, D), jnp.float32)
                  for _ in range(NE)],
        mesh=_sc_mesh(),
        scratch_types=_sc_scratch(),
    )
    def agg(v0_hbm, v1_hbm, v2_hbm, v3_hbm, src_hbm, dst_hbm, zrow_hbm,
            o0_hbm, o1_hbm, o2_hbm, o3_hbm,
            rb0, rb1, sb0, sb1, db0, db1,
            sg0, sg1, si0, si1, sd0, sd1, ss0, ss1, acc_sh):
        c = lax.axis_index("c")
        s = lax.axis_index("s")
        n = jnp.int32(n0) + c * jnp.int32(n1 - n0)
        T = jnp.int32(n0 // 2) + c * jnp.int32(n1 // 2 - n0 // 2)
        base = s * (n0 + n1) + c * n0
        r0 = s * ROWS_PER_TILE
        o0 = c * N_PAD + r0
        rbs = [rb0, rb1]
        vs = [v0_hbm, v1_hbm, v2_hbm, v3_hbm]
        os_ = [o0_hbm, o1_hbm, o2_hbm, o3_hbm]
        for e in range(NE):
            ops = _make_ring_ops(base, n, vs[e], src_hbm, dst_hbm, rbs,
                                 [sb0, sb1], [db0, db1],
                                 [sg0, sg1], [si0, si1],
                                 [sd0, sd1], [ss0, ss1],
                                 acc_sh)
            pltpu.sync_copy(zrow_hbm, rbs[0])
            for j in range(RCH):
                pltpu.sync_copy(rbs[0], acc_sh.at[pl.ds(r0 + j * CH, CH)])
            plsc.subcore_barrier()
            _sc_pipeline(n, T, ops["s_wait"], ops["id_issue"],
                         ops["id_wait"], ops["s_issue"], ops["g_issue"],
                         ops["g_wait"], ops["is_issue"], ops["is_wait"],
                         R=2, d=1)
            plsc.subcore_barrier()
            for j in range(RCH):
                pltpu.sync_copy(acc_sh.at[pl.ds(r0 + j * CH, CH)], rbs[0])
                pltpu.sync_copy(rbs[0], os_[e].at[pl.ds(o0 + j * CH, CH)])

    return agg


def _encoder_body(x_ref, w_ref, b_ref, bfull_ref, bblk_ref, cent_ref,
                  h_ref, p_ref):
    h = jnp.dot(x_ref[...], w_ref[...], preferred_element_type=jnp.float32)
    h_ref[...] = jnp.maximum(h + b_ref[...], 0.0)
    bf = bfull_ref[...]
    bb = bblk_ref[...]
    inv_logn = 1.0 / jnp.log(jnp.float32(N))
    logn = jnp.zeros((RB, 1), jnp.float32)
    for g in range(NG):
        cnt = jnp.sum(jnp.where(bf == g, 1.0, 0.0))
        lg = jnp.log(jnp.maximum(cnt, 1.0)) * inv_logn
        logn = logn + jnp.where(bb == g, lg, 0.0)
    dlt = logn - cent_ref[...]
    sc = -(dlt * dlt)
    m = jnp.max(sc, axis=1, keepdims=True)
    ex = jnp.exp(sc - m)
    p_ref[...] = ex / jnp.sum(ex, axis=1, keepdims=True)


def _layer1_body(h_ref, acc_ref, deg_ref, ws_ref, wn_ref, b_ref,
                 o0_ref, o1_ref, o2_ref, o3_ref):
    dg = deg_ref[:, 0:1]
    inv = 1.0 / jnp.maximum(dg, 1.0)
    m1 = acc_ref[...] * inv
    h = h_ref[...]
    outs = [o0_ref, o1_ref, o2_ref, o3_ref]
    for e in range(NE):
        ye = (jnp.dot(h, ws_ref[e], preferred_element_type=jnp.float32)
              + jnp.dot(m1, wn_ref[e], preferred_element_type=jnp.float32)
              + b_ref[e:e + 1, :])
        outs[e][...] = jnp.maximum(ye, 0.0)


def _layer2_body(h0_ref, h1_ref, h2_ref, h3_ref, a0_ref, a1_ref, a2_ref,
                 a3_ref, deg_ref, p_ref, ws_ref, wn_ref, b_ref, out_ref):
    dg = deg_ref[:, 0:1]
    inv = 1.0 / jnp.maximum(dg, 1.0)
    p = p_ref[...]
    out = jnp.zeros((RB, D), jnp.float32)
    hes = [h0_ref, h1_ref, h2_ref, h3_ref]
    accs = [a0_ref, a1_ref, a2_ref, a3_ref]
    for e in range(NE):
        m2 = (accs[e][0] + accs[e][1]) * inv
        ye = (jnp.dot(hes[e][...], ws_ref[e], preferred_element_type=jnp.float32)
              + jnp.dot(m2, wn_ref[e], preferred_element_type=jnp.float32)
              + b_ref[e:e + 1, :])
        out = out + p[:, e:e + 1] * ye
    out_ref[...] = out


def kernel(x, edge_index, batch, W_enc, b_enc, Wself1, Wneigh1, b1,
           Wself2, Wneigh2, b2, centers):
    src = edge_index[0].astype(jnp.int32)
    dst = edge_index[1].astype(jnp.int32)
    e_edges = src.shape[0]
    m_pair = 2 * (-(-e_edges // (NS * CH * 2)))
    n0_c = max(8, 2 * int(round(m_pair * 0.825 / 2.0)))
    n1_c = m_pair - n0_c
    e_pad = m_pair * NS * CH
    npad = e_pad - e_edges
    src_p = jnp.concatenate(
        [src, jnp.zeros((npad,), jnp.int32)])
    dst_p = jnp.concatenate(
        [dst, N + (jnp.arange(npad, dtype=jnp.int32) % CH)])
    zrow = jnp.zeros((CH, D), jnp.float32)
    ones128 = jnp.ones((CH, D), jnp.float32)

    batch_i = batch.astype(jnp.int32)
    batch_full = jnp.concatenate(
        [batch_i, jnp.full((NB_PAD - N,), NG, jnp.int32)]).reshape(1, NB_PAD)
    batch_blk = batch_i.reshape(N, 1)

    h, probs = pl.pallas_call(
        _encoder_body,
        grid=(GRID,),
        in_specs=[
            pl.BlockSpec((RB, D), lambda i: (i, 0)),
            pl.BlockSpec((D, D), lambda i: (0, 0)),
            pl.BlockSpec((1, D), lambda i: (0, 0)),
            pl.BlockSpec((1, NB_PAD), lambda i: (0, 0)),
            pl.BlockSpec((RB, 1), lambda i: (i, 0)),
            pl.BlockSpec((1, NE), lambda i: (0, 0)),
        ],
        out_specs=[
            pl.BlockSpec((RB, D), lambda i: (i, 0)),
            pl.BlockSpec((RB, NE), lambda i: (i, 0)),
        ],
        out_shape=[
            jax.ShapeDtypeStruct((N, D), jnp.float32),
            jax.ShapeDtypeStruct((N, NE), jnp.float32),
        ],
    )(x, W_enc, b_enc.reshape(1, D), batch_full, batch_blk,
      centers.reshape(1, NE))

    acc1, deg = _make_agg_h(m_pair)(h, src_p, dst_p, zrow, ones128)

    hes = pl.pallas_call(
        _layer1_body,
        grid=(GRID,),
        in_specs=[
            pl.BlockSpec((RB, D), lambda i: (i, 0)),
            pl.BlockSpec((RB, D), lambda i: (i, 0)),
            pl.BlockSpec((RB, D), lambda i: (i, 0)),
            pl.BlockSpec((NE, D, D), lambda i: (0, 0, 0)),
            pl.BlockSpec((NE, D, D), lambda i: (0, 0, 0)),
            pl.BlockSpec((NE, D), lambda i: (0, 0)),
        ],
        out_specs=[pl.BlockSpec((RB, D), lambda i: (i, 0))
                   for _ in range(NE)],
        out_shape=[jax.ShapeDtypeStruct((N, D), jnp.float32)
                   for _ in range(NE)],
    )(h, acc1, deg, Wself1, Wneigh1, b1)

    acc2_fs = _make_agg_experts(n0_c, n1_c)(
        hes[0], hes[1], hes[2], hes[3], src_p, dst_p, zrow)
    acc2s = [a.reshape(NC, N_PAD, D) for a in acc2_fs]

    out = pl.pallas_call(
        _layer2_body,
        grid=(GRID,),
        in_specs=(
            [pl.BlockSpec((RB, D), lambda i: (i, 0)) for _ in range(NE)]
            + [pl.BlockSpec((NC, RB, D), lambda i: (0, i, 0))
               for _ in range(NE)]
            + [
                pl.BlockSpec((RB, D), lambda i: (i, 0)),
                pl.BlockSpec((RB, NE), lambda i: (i, 0)),
                pl.BlockSpec((NE, D, D), lambda i: (0, 0, 0)),
                pl.BlockSpec((NE, D, D), lambda i: (0, 0, 0)),
                pl.BlockSpec((NE, D), lambda i: (0, 0)),
            ]
        ),
        out_specs=pl.BlockSpec((RB, D), lambda i: (i, 0)),
        out_shape=jax.ShapeDtypeStruct((N, D), jnp.float32),
    )(hes[0], hes[1], hes[2], hes[3], acc2s[0], acc2s[1], acc2s[2],
      acc2s[3], deg, probs, Wself2, Wneigh2, b2)
    return out
